# baseline (device time: 10301 ns/iter reference)
import functools
import os

import jax
import jax.numpy as jnp
from jax import lax
from jax.experimental import pallas as pl
from jax.experimental.pallas import tpu as pltpu

N_DEV = 16
_MODE = os.environ.get("KERNEL_MODE", "full")


def _mod2(c):
    return c - 2.0 * jnp.floor(c * 0.5)


def kernel(x):
    m, n = x.shape

    if _MODE == "compute":
        def probe(x_ref, out_ref):
            xv = x_ref[:, :]
            la = jnp.log(jnp.abs(xv))
            neg = jnp.where(xv < 0.0, 1.0, 0.0)
            b = jnp.concatenate([la, neg], axis=1)
            r = lax.broadcasted_iota(jnp.int32, (m, m), 0)
            c = lax.broadcasted_iota(jnp.int32, (m, m), 1)
            tril = jnp.where(r >= c, 1.0, 0.0)
            s = jax.lax.dot(tril, b)
            out_ref[:, :] = jnp.exp(s[:, :n]) * (1.0 - 2.0 * _mod2(s[:, n:]))

        return pl.pallas_call(
            probe,
            out_shape=jax.ShapeDtypeStruct((m, n), jnp.float32),
            in_specs=[pl.BlockSpec(memory_space=pltpu.VMEM)],
            out_specs=pl.BlockSpec(memory_space=pltpu.VMEM),
        )(x)

    def body(x_hbm, out_hbm, comm_ref, send_buf,
             xv_buf, ov_buf, send_sems, recv_sems, credit_sems, dma_sems):
        my = lax.axis_index("i")

        cp_in = pltpu.make_async_copy(x_hbm, xv_buf, dma_sems.at[0])
        cp_in.start()

        barrier = pltpu.get_barrier_semaphore()
        pl.semaphore_signal(
            barrier, inc=1,
            device_id=(my,), device_id_type=pl.DeviceIdType.MESH,
        )
        pl.semaphore_wait(barrier, 1)

        for j in range(N_DEV):
            @pl.when(j < my)
            def _():
                pl.semaphore_signal(
                    credit_sems.at[my], inc=1,
                    device_id=(j,), device_id_type=pl.DeviceIdType.MESH,
                )

        cp_in.wait()

        xv = xv_buf[:, :]
        la = jnp.log(jnp.abs(xv))
        neg = jnp.where(xv < 0.0, 1.0, 0.0)
        tot = jnp.concatenate(
            [jnp.sum(la, axis=0, keepdims=True),
             jnp.sum(neg, axis=0, keepdims=True)],
            axis=1,
        )
        send_buf[:, :] = tot

        for k in range(N_DEV):
            @pl.when(my < k)
            def _():
                pl.semaphore_wait(credit_sems.at[k], 1)
                rdma = pltpu.make_async_remote_copy(
                    src_ref=send_buf,
                    dst_ref=comm_ref.at[my],
                    send_sem=send_sems.at[k],
                    recv_sem=recv_sems.at[my],
                    device_id=(k,),
                    device_id_type=pl.DeviceIdType.MESH,
                )
                rdma.start()

        if _MODE == "comm":
            local = xv
        else:
            b = jnp.concatenate([la, neg], axis=1)
            r = lax.broadcasted_iota(jnp.int32, (m, m), 0)
            c = lax.broadcasted_iota(jnp.int32, (m, m), 1)
            tril = jnp.where(r >= c, 1.0, 0.0)
            s = jax.lax.dot(tril, b)
            local = jnp.exp(s[:, :n]) * (1.0 - 2.0 * _mod2(s[:, n:]))

        for j in range(N_DEV):
            @pl.when(j < my)
            def _():
                recv = pltpu.make_async_remote_copy(
                    src_ref=send_buf,
                    dst_ref=comm_ref.at[j],
                    send_sem=send_sems.at[j],
                    recv_sem=recv_sems.at[j],
                    device_id=(j,),
                    device_id_type=pl.DeviceIdType.MESH,
                )
                recv.wait_recv()

        p = jnp.zeros((1, 2 * n), jnp.float32)
        for j in range(N_DEV):
            p = p + jnp.where(j < my, comm_ref[j], 0.0)
        scale = jnp.exp(p[:, :n]) * (1.0 - 2.0 * _mod2(p[:, n:]))
        ov_buf[:, :] = local * scale

        cp_out = pltpu.make_async_copy(ov_buf, out_hbm, dma_sems.at[1])
        cp_out.start()

        for k in range(N_DEV):
            @pl.when(my < k)
            def _():
                rdma = pltpu.make_async_remote_copy(
                    src_ref=send_buf,
                    dst_ref=comm_ref.at[my],
                    send_sem=send_sems.at[k],
                    recv_sem=recv_sems.at[my],
                    device_id=(k,),
                    device_id_type=pl.DeviceIdType.MESH,
                )
                rdma.wait_send()

        cp_out.wait()

    return pl.pallas_call(
        body,
        out_shape=jax.ShapeDtypeStruct((m, n), jnp.float32),
        in_specs=[pl.BlockSpec(memory_space=pltpu.HBM)],
        out_specs=pl.BlockSpec(memory_space=pltpu.HBM),
        scratch_shapes=[
            pltpu.VMEM((N_DEV, 1, 2 * n), jnp.float32),
            pltpu.VMEM((1, 2 * n), jnp.float32),
            pltpu.VMEM((m, n), jnp.float32),
            pltpu.VMEM((m, n), jnp.float32),
            pltpu.SemaphoreType.DMA((N_DEV,)),
            pltpu.SemaphoreType.DMA((N_DEV,)),
            pltpu.SemaphoreType.REGULAR((N_DEV,)),
            pltpu.SemaphoreType.DMA((2,)),
        ],
        compiler_params=pltpu.CompilerParams(collective_id=0),
    )(x)


# device time: 10127 ns/iter; 1.0172x vs baseline; 1.0172x over previous
import functools
import os

import jax
import jax.numpy as jnp
from jax import lax
from jax.experimental import pallas as pl
from jax.experimental.pallas import tpu as pltpu

N_DEV = 16
_MODE = os.environ.get("KERNEL_MODE", "full")


def _mod2(c):
    return c - 2.0 * jnp.floor(c * 0.5)


def kernel(x):
    m, n = x.shape

    if _MODE == "compute":
        def probe(x_ref, out_ref):
            xv = x_ref[:, :]
            la = jnp.log(jnp.abs(xv))
            neg = jnp.where(xv < 0.0, 1.0, 0.0)
            b = jnp.concatenate([la, neg], axis=1)
            r = lax.broadcasted_iota(jnp.int32, (m, m), 0)
            c = lax.broadcasted_iota(jnp.int32, (m, m), 1)
            tril = jnp.where(r >= c, 1.0, 0.0)
            s = jax.lax.dot(tril, b)
            out_ref[:, :] = jnp.exp(s[:, :n]) * (1.0 - 2.0 * _mod2(s[:, n:]))

        return pl.pallas_call(
            probe,
            out_shape=jax.ShapeDtypeStruct((m, n), jnp.float32),
            in_specs=[pl.BlockSpec(memory_space=pltpu.VMEM)],
            out_specs=pl.BlockSpec(memory_space=pltpu.VMEM),
        )(x)

    def body(x_ref, out_ref, comm_ref, send_buf,
             send_sems, recv_sems, credit_sems):
        my = lax.axis_index("i")

        barrier = pltpu.get_barrier_semaphore()
        pl.semaphore_signal(
            barrier, inc=1,
            device_id=(my,), device_id_type=pl.DeviceIdType.MESH,
        )
        pl.semaphore_wait(barrier, 1)

        for j in range(N_DEV):
            @pl.when(j < my)
            def _():
                pl.semaphore_signal(
                    credit_sems.at[my], inc=1,
                    device_id=(j,), device_id_type=pl.DeviceIdType.MESH,
                )

        xv = x_ref[:, :]
        la = jnp.log(jnp.abs(xv))
        neg = jnp.where(xv < 0.0, 1.0, 0.0)
        tot = jnp.concatenate(
            [jnp.sum(la, axis=0, keepdims=True),
             jnp.sum(neg, axis=0, keepdims=True)],
            axis=1,
        )
        send_buf[:, :] = tot

        for k in range(N_DEV):
            @pl.when(my < k)
            def _():
                pl.semaphore_wait(credit_sems.at[k], 1)
                rdma = pltpu.make_async_remote_copy(
                    src_ref=send_buf,
                    dst_ref=comm_ref.at[my],
                    send_sem=send_sems.at[k],
                    recv_sem=recv_sems.at[my],
                    device_id=(k,),
                    device_id_type=pl.DeviceIdType.MESH,
                )
                rdma.start()

        if _MODE == "comm":
            local = xv
        else:
            b = jnp.concatenate([la, neg], axis=1)
            r = lax.broadcasted_iota(jnp.int32, (m, m), 0)
            c = lax.broadcasted_iota(jnp.int32, (m, m), 1)
            tril = jnp.where(r >= c, 1.0, 0.0)
            s = jax.lax.dot(tril, b)
            local = jnp.exp(s[:, :n]) * (1.0 - 2.0 * _mod2(s[:, n:]))

        for j in range(N_DEV):
            @pl.when(j < my)
            def _():
                recv = pltpu.make_async_remote_copy(
                    src_ref=send_buf,
                    dst_ref=comm_ref.at[j],
                    send_sem=send_sems.at[j],
                    recv_sem=recv_sems.at[j],
                    device_id=(j,),
                    device_id_type=pl.DeviceIdType.MESH,
                )
                recv.wait_recv()

        p = jnp.zeros((1, 2 * n), jnp.float32)
        for j in range(N_DEV):
            p = p + jnp.where(j < my, comm_ref[j], 0.0)
        scale = jnp.exp(p[:, :n]) * (1.0 - 2.0 * _mod2(p[:, n:]))
        out_ref[:, :] = local * scale

        for k in range(N_DEV):
            @pl.when(my < k)
            def _():
                rdma = pltpu.make_async_remote_copy(
                    src_ref=send_buf,
                    dst_ref=comm_ref.at[my],
                    send_sem=send_sems.at[k],
                    recv_sem=recv_sems.at[my],
                    device_id=(k,),
                    device_id_type=pl.DeviceIdType.MESH,
                )
                rdma.wait_send()

    return pl.pallas_call(
        body,
        out_shape=jax.ShapeDtypeStruct((m, n), jnp.float32),
        in_specs=[pl.BlockSpec(memory_space=pltpu.VMEM)],
        out_specs=pl.BlockSpec(memory_space=pltpu.VMEM),
        scratch_shapes=[
            pltpu.VMEM((N_DEV, 1, 2 * n), jnp.float32),
            pltpu.VMEM((1, 2 * n), jnp.float32),
            pltpu.SemaphoreType.DMA((N_DEV,)),
            pltpu.SemaphoreType.DMA((N_DEV,)),
            pltpu.SemaphoreType.REGULAR((N_DEV,)),
        ],
        compiler_params=pltpu.CompilerParams(collective_id=0),
    )(x)


# device time: 9927 ns/iter; 1.0377x vs baseline; 1.0201x over previous
import functools
import os

import jax
import jax.numpy as jnp
from jax import lax
from jax.experimental import pallas as pl
from jax.experimental.pallas import tpu as pltpu

N_DEV = 16
_MODE = os.environ.get("KERNEL_MODE", "full")


def _mod2(c):
    return c - 2.0 * jnp.floor(c * 0.5)


def kernel(x):
    m, n = x.shape

    if _MODE == "compute":
        def probe(x_ref, out_ref):
            xv = x_ref[:, :]
            la = jnp.log(jnp.abs(xv))
            neg = jnp.where(xv < 0.0, 1.0, 0.0)
            b = jnp.concatenate([la, neg], axis=1)
            r = lax.broadcasted_iota(jnp.int32, (m, m), 0)
            c = lax.broadcasted_iota(jnp.int32, (m, m), 1)
            tril = jnp.where(r >= c, 1.0, 0.0)
            s = jax.lax.dot(tril, b)
            out_ref[:, :] = jnp.exp(s[:, :n]) * (1.0 - 2.0 * _mod2(s[:, n:]))

        return pl.pallas_call(
            probe,
            out_shape=jax.ShapeDtypeStruct((m, n), jnp.float32),
            in_specs=[pl.BlockSpec(memory_space=pltpu.VMEM)],
            out_specs=pl.BlockSpec(memory_space=pltpu.VMEM),
        )(x)

    def body(x_ref, out_ref, comm_ref, send_buf,
             send_sems, recv_sems, credit_sems):
        my = lax.axis_index("i")

        barrier = pltpu.get_barrier_semaphore()
        pl.semaphore_signal(
            barrier, inc=1,
            device_id=(my,), device_id_type=pl.DeviceIdType.MESH,
        )
        pl.semaphore_wait(barrier, 1)

        for j in range(N_DEV):
            @pl.when(j < my)
            def _():
                pl.semaphore_signal(
                    credit_sems.at[my], inc=1,
                    device_id=(j,), device_id_type=pl.DeviceIdType.MESH,
                )

        xv = x_ref[:, :]
        la = jnp.log(jnp.abs(xv))
        neg = jnp.where(xv < 0.0, 1.0, 0.0)
        tot = jnp.concatenate(
            [jnp.sum(la, axis=0, keepdims=True),
             jnp.sum(neg, axis=0, keepdims=True)],
            axis=1,
        )
        send_buf[:, :] = tot

        def _send_to(k):
            pl.semaphore_wait(credit_sems.at[k], 1)
            rdma = pltpu.make_async_remote_copy(
                src_ref=send_buf,
                dst_ref=comm_ref.at[my],
                send_sem=send_sems.at[k],
                recv_sem=recv_sems.at[my],
                device_id=(k,),
                device_id_type=pl.DeviceIdType.MESH,
            )
            rdma.start()

        for k in range(N_DEV):
            @pl.when(jnp.logical_and(my < k, k <= my + 3))
            def _():
                _send_to(k)

        if _MODE == "comm":
            local = xv
        else:
            b = jnp.concatenate([la, neg], axis=1)
            r = lax.broadcasted_iota(jnp.int32, (m, m), 0)
            c = lax.broadcasted_iota(jnp.int32, (m, m), 1)
            tril = jnp.where(r >= c, 1.0, 0.0)
            s = jax.lax.dot(tril, b)
            local = jnp.exp(s[:, :n]) * (1.0 - 2.0 * _mod2(s[:, n:]))

        for k in range(N_DEV):
            @pl.when(k > my + 3)
            def _():
                _send_to(k)

        for j in range(N_DEV):
            @pl.when(j < my)
            def _():
                recv = pltpu.make_async_remote_copy(
                    src_ref=send_buf,
                    dst_ref=comm_ref.at[j],
                    send_sem=send_sems.at[j],
                    recv_sem=recv_sems.at[j],
                    device_id=(j,),
                    device_id_type=pl.DeviceIdType.MESH,
                )
                recv.wait_recv()

        p = jnp.zeros((1, 2 * n), jnp.float32)
        for j in range(N_DEV):
            p = p + jnp.where(j < my, comm_ref[j], 0.0)
        scale = jnp.exp(p[:, :n]) * (1.0 - 2.0 * _mod2(p[:, n:]))
        out_ref[:, :] = local * scale

        for k in range(N_DEV):
            @pl.when(my < k)
            def _():
                rdma = pltpu.make_async_remote_copy(
                    src_ref=send_buf,
                    dst_ref=comm_ref.at[my],
                    send_sem=send_sems.at[k],
                    recv_sem=recv_sems.at[my],
                    device_id=(k,),
                    device_id_type=pl.DeviceIdType.MESH,
                )
                rdma.wait_send()

    return pl.pallas_call(
        body,
        out_shape=jax.ShapeDtypeStruct((m, n), jnp.float32),
        in_specs=[pl.BlockSpec(memory_space=pltpu.VMEM)],
        out_specs=pl.BlockSpec(memory_space=pltpu.VMEM),
        scratch_shapes=[
            pltpu.VMEM((N_DEV, 1, 2 * n), jnp.float32),
            pltpu.VMEM((1, 2 * n), jnp.float32),
            pltpu.SemaphoreType.DMA((N_DEV,)),
            pltpu.SemaphoreType.DMA((N_DEV,)),
            pltpu.SemaphoreType.REGULAR((N_DEV,)),
        ],
        compiler_params=pltpu.CompilerParams(collective_id=0),
    )(x)


# device time: 9910 ns/iter; 1.0395x vs baseline; 1.0017x over previous
import functools
import os

import jax
import jax.numpy as jnp
from jax import lax
from jax.experimental import pallas as pl
from jax.experimental.pallas import tpu as pltpu

N_DEV = 16
_MODE = os.environ.get("KERNEL_MODE", "full")


def _mod2(c):
    return c - 2.0 * jnp.floor(c * 0.5)


def kernel(x):
    m, n = x.shape

    if _MODE == "compute":
        def probe(x_ref, out_ref):
            xv = x_ref[:, :]
            la = jnp.log(jnp.abs(xv))
            neg = jnp.where(xv < 0.0, 1.0, 0.0)
            b = jnp.concatenate([la, neg], axis=1)
            r = lax.broadcasted_iota(jnp.int32, (m, m), 0)
            c = lax.broadcasted_iota(jnp.int32, (m, m), 1)
            tril = jnp.where(r >= c, 1.0, 0.0)
            s = jax.lax.dot(tril, b)
            out_ref[:, :] = jnp.exp(s[:, :n]) * (1.0 - 2.0 * _mod2(s[:, n:]))

        return pl.pallas_call(
            probe,
            out_shape=jax.ShapeDtypeStruct((m, n), jnp.float32),
            in_specs=[pl.BlockSpec(memory_space=pltpu.VMEM)],
            out_specs=pl.BlockSpec(memory_space=pltpu.VMEM),
        )(x)

    def body(x_ref, out_ref, comm_ref, send_buf,
             send_sems, recv_sems, credit_sems):
        my = lax.axis_index("i")

        barrier = pltpu.get_barrier_semaphore()
        pl.semaphore_signal(
            barrier, inc=1,
            device_id=(my,), device_id_type=pl.DeviceIdType.MESH,
        )
        pl.semaphore_wait(barrier, 1)

        for j in range(N_DEV):
            @pl.when(j < my)
            def _():
                pl.semaphore_signal(
                    credit_sems.at[my], inc=1,
                    device_id=(j,), device_id_type=pl.DeviceIdType.MESH,
                )

        xv = x_ref[:, :]
        la = jnp.log(jnp.abs(xv))
        neg = jnp.where(xv < 0.0, 1.0, 0.0)
        tot = jnp.concatenate(
            [jnp.sum(la, axis=0, keepdims=True),
             jnp.sum(neg, axis=0, keepdims=True)],
            axis=1,
        )
        send_buf[:, :] = tot

        def _send_to(k):
            pl.semaphore_wait(credit_sems.at[k], 1)
            rdma = pltpu.make_async_remote_copy(
                src_ref=send_buf,
                dst_ref=comm_ref.at[my],
                send_sem=send_sems.at[k],
                recv_sem=recv_sems.at[my],
                device_id=(k,),
                device_id_type=pl.DeviceIdType.MESH,
            )
            rdma.start()

        for k in range(N_DEV):
            @pl.when(jnp.logical_and(my < k, k <= my + 3))
            def _():
                _send_to(k)

        if _MODE == "comm":
            local = xv
        else:
            b = jnp.concatenate([la, neg], axis=1)
            r = lax.broadcasted_iota(jnp.int32, (m, m), 0)
            c = lax.broadcasted_iota(jnp.int32, (m, m), 1)
            tril = jnp.where(r >= c, 1.0, 0.0)
            s = jax.lax.dot(tril, b)
            local = jnp.exp(s[:, :n]) * (1.0 - 2.0 * _mod2(s[:, n:]))

        for k in range(N_DEV):
            @pl.when(k > my + 3)
            def _():
                _send_to(k)

        for j in range(N_DEV):
            @pl.when(j < my)
            def _():
                recv = pltpu.make_async_remote_copy(
                    src_ref=send_buf,
                    dst_ref=comm_ref.at[j],
                    send_sem=send_sems.at[j],
                    recv_sem=recv_sems.at[j],
                    device_id=(j,),
                    device_id_type=pl.DeviceIdType.MESH,
                )
                recv.wait_recv()

        all_tots = comm_ref[:, 0, :]
        src = lax.broadcasted_iota(jnp.int32, (N_DEV, 2 * n), 0)
        p = jnp.sum(
            jnp.where(src < my, all_tots, 0.0), axis=0, keepdims=True
        )
        scale = jnp.exp(p[:, :n]) * (1.0 - 2.0 * _mod2(p[:, n:]))
        out_ref[:, :] = local * scale

        for k in range(N_DEV):
            @pl.when(my < k)
            def _():
                rdma = pltpu.make_async_remote_copy(
                    src_ref=send_buf,
                    dst_ref=comm_ref.at[my],
                    send_sem=send_sems.at[k],
                    recv_sem=recv_sems.at[my],
                    device_id=(k,),
                    device_id_type=pl.DeviceIdType.MESH,
                )
                rdma.wait_send()

    return pl.pallas_call(
        body,
        out_shape=jax.ShapeDtypeStruct((m, n), jnp.float32),
        in_specs=[pl.BlockSpec(memory_space=pltpu.VMEM)],
        out_specs=pl.BlockSpec(memory_space=pltpu.VMEM),
        scratch_shapes=[
            pltpu.VMEM((N_DEV, 1, 2 * n), jnp.float32),
            pltpu.VMEM((1, 2 * n), jnp.float32),
            pltpu.SemaphoreType.DMA((N_DEV,)),
            pltpu.SemaphoreType.DMA((N_DEV,)),
            pltpu.SemaphoreType.REGULAR((N_DEV,)),
        ],
        compiler_params=pltpu.CompilerParams(collective_id=0),
    )(x)
